# baseline (speedup 1.0000x reference)
import functools

import jax
import jax.numpy as jnp
from jax import lax
from jax.experimental import pallas as pl
from jax.experimental.pallas import tpu as pltpu
from jax.experimental.pallas import tpu_sc as plsc

NUM_FEATURES = 9
NUM_BINS = 101
EMBED_DIM = 512
BATCH = 16384

_ROWS = BATCH * NUM_FEATURES
_NW = 32
_CHUNK = 48
_NBUF = 3
_LANES = 16
_MAGIC = 8388608.0

_SC_SLOTS = 24
_ROWS_SC = _NW * _CHUNK * _SC_SLOTS
_R1 = _ROWS - _ROWS_SC
_RPW = _CHUNK * _SC_SLOTS
_TCB = 512
_KPAD = 1024


def _sc_gather(feat_flat, w_flat):
    mesh = plsc.VectorSubcoreMesh(core_axis_name="c", subcore_axis_name="s")

    @functools.partial(
        pl.kernel,
        mesh=mesh,
        out_type=jax.ShapeDtypeStruct((_ROWS, EMBED_DIM), jnp.float32),
        scratch_types=[
            pltpu.VMEM_SHARED((NUM_FEATURES * NUM_BINS, EMBED_DIM),
                              jnp.float32),
            pltpu.VMEM((_RPW,), jnp.float32),
            *[pltpu.VMEM((_CHUNK,), jnp.int32) for _ in range(_NBUF)],
            *[pltpu.VMEM((_CHUNK, EMBED_DIM), jnp.float32)
              for _ in range(_NBUF)],
            *[pltpu.SemaphoreType.DMA for _ in range(2 * _NBUF)],
        ],
    )
    def body(feat_hbm, w_hbm, out_hbm, w_sh, feat_v, *scratch):
        idx_v = scratch[:_NBUF]
        rows_v = scratch[_NBUF:2 * _NBUF]
        g_sem = scratch[2 * _NBUF:3 * _NBUF]
        o_sem = scratch[3 * _NBUF:]

        wid = lax.axis_index("s") * 2 + lax.axis_index("c")
        w_base = _R1 + wid * _RPW
        iota = lax.iota(jnp.int32, _LANES)

        @pl.when(lax.axis_index("s") == 0)
        def _():
            pltpu.sync_copy(w_hbm, w_sh)

        plsc.subcore_barrier()
        pltpu.sync_copy(feat_hbm.at[pl.ds(w_base, _RPW)], feat_v)

        def start_fetch(k, b):
            for i in range(_CHUNK // _LANES):
                off = k * _CHUNK + i * _LANES
                x = feat_v[pl.ds(off, _LANES)]
                xc = jnp.minimum(jnp.maximum(x, 0.0), float(NUM_BINS - 1))
                r = (xc + _MAGIC) - _MAGIC
                fid = lax.rem(w_base + off + iota, jnp.int32(NUM_FEATURES))
                idx_v[b][pl.ds(i * _LANES, _LANES)] = (
                    fid * NUM_BINS + r.astype(jnp.int32))

            def fire(n, carry):
                v = idx_v[b][pl.ds(n * _LANES, _LANES)]
                for u in range(_LANES):
                    pltpu.async_copy(
                        w_sh.at[pl.ds(v[u], 1)],
                        rows_v[b].at[pl.ds(n * _LANES + u, 1)],
                        g_sem[b])
                return carry

            lax.fori_loop(0, _CHUNK // _LANES, fire, 0)

        def wait_fetch(b):
            pltpu.make_async_copy(
                w_hbm.at[pl.ds(0, _CHUNK)], rows_v[b], g_sem[b]).wait()

        start_fetch(jnp.int32(0), 0)

        def round_body(rnd, carry):
            for b in range(_NBUF):
                j = rnd * _NBUF + b
                wait_fetch(b)
                pltpu.async_copy(
                    rows_v[b],
                    out_hbm.at[pl.ds(w_base + j * _CHUNK, _CHUNK)],
                    o_sem[b])
                k = j + 1
                b2 = (b + 1) % _NBUF

                @pl.when(k < _SC_SLOTS)
                def _():
                    @pl.when(k >= _NBUF)
                    def _():
                        pltpu.make_async_copy(
                            rows_v[b2],
                            out_hbm.at[
                                pl.ds(w_base + (k - _NBUF) * _CHUNK, _CHUNK)],
                            o_sem[b2]).wait()

                    start_fetch(k, b2)

            return carry

        lax.fori_loop(0, _SC_SLOTS // _NBUF, round_body, 0)

        for b in range(_NBUF):
            j = _SC_SLOTS - _NBUF + b
            pltpu.make_async_copy(
                rows_v[b],
                out_hbm.at[pl.ds(w_base + j * _CHUNK, _CHUNK)],
                o_sem[b]).wait()

    return body(feat_flat, w_flat)


def _tc_fill(feat3, wpad, sc_out):
    ntc = _R1 // _TCB

    def tck(feat_ref, w_ref, prev_ref, out_ref):
        del prev_ref
        g = pl.program_id(0)
        x = feat_ref[0]
        xc = jnp.minimum(jnp.maximum(x, 0.0), float(NUM_BINS - 1))
        r = (xc + _MAGIC) - _MAGIC
        e = g * _TCB + lax.broadcasted_iota(jnp.int32, (_TCB, 1), 0)
        flat = (lax.rem(e, jnp.int32(NUM_FEATURES)) * NUM_BINS
                + r.astype(jnp.int32))
        oh = (lax.broadcasted_iota(jnp.int32, (_TCB, _KPAD), 1) == flat
              ).astype(jnp.bfloat16)
        out_ref[...] = lax.dot_general(
            oh, w_ref[...], (((1,), (0,)), ((), ())),
            preferred_element_type=jnp.float32)

    return pl.pallas_call(
        tck,
        grid=(ntc,),
        in_specs=[
            pl.BlockSpec((1, _TCB, 1), lambda i: (i, 0, 0)),
            pl.BlockSpec((_KPAD, EMBED_DIM), lambda i: (0, 0)),
            pl.BlockSpec(memory_space=pltpu.MemorySpace.HBM),
        ],
        out_specs=pl.BlockSpec((_TCB, EMBED_DIM), lambda i: (i, 0)),
        out_shape=jax.ShapeDtypeStruct((_ROWS, EMBED_DIM), jnp.float32),
        input_output_aliases={2: 0},
    )(feat3, wpad, sc_out)


def kernel(features, W):
    feat_flat = features.reshape(_ROWS)
    w_flat = W.reshape(NUM_FEATURES * NUM_BINS, EMBED_DIM)

    sc_out = _sc_gather(feat_flat, w_flat)

    feat3 = feat_flat[:_R1].reshape(_R1 // _TCB, _TCB, 1)
    wpad = jnp.zeros((_KPAD, EMBED_DIM), jnp.bfloat16)
    wpad = wpad.at[:NUM_FEATURES * NUM_BINS].set(w_flat.astype(jnp.bfloat16))
    out = _tc_fill(feat3, wpad, sc_out)
    return out.reshape(BATCH, NUM_FEATURES * EMBED_DIM)


# speedup vs baseline: 1.3392x; 1.3392x over previous
import functools

import jax
import jax.numpy as jnp
from jax import lax
from jax.experimental import pallas as pl
from jax.experimental.pallas import tpu as pltpu
from jax.experimental.pallas import tpu_sc as plsc

NUM_FEATURES = 9
NUM_BINS = 101
EMBED_DIM = 512
BATCH = 16384

_ROWS = BATCH * NUM_FEATURES
_NW = 32
_ROWS_PER_W = _ROWS // _NW
_CHUNK = 32
_NCHUNK = _ROWS_PER_W // _CHUNK
_NBUF = 4
_LOOKAHEAD = 2
_LANES = 16
_MAGIC = 8388608.0


def _sc_gather(feat_flat, w_flat):
    mesh = plsc.VectorSubcoreMesh(core_axis_name="c", subcore_axis_name="s")

    @functools.partial(
        pl.kernel,
        mesh=mesh,
        out_type=jax.ShapeDtypeStruct((_ROWS, EMBED_DIM), jnp.float32),
        scratch_types=[
            pltpu.VMEM_SHARED((NUM_FEATURES * NUM_BINS, EMBED_DIM),
                              jnp.float32),
            pltpu.VMEM((_ROWS_PER_W,), jnp.float32),
            *[pltpu.VMEM((_CHUNK,), jnp.int32) for _ in range(_NBUF)],
            *[pltpu.VMEM((_CHUNK, EMBED_DIM), jnp.float32)
              for _ in range(_NBUF)],
            *[pltpu.SemaphoreType.DMA for _ in range(2 * _NBUF)],
        ],
    )
    def body(feat_hbm, w_hbm, out_hbm, w_sh, feat_v, *scratch):
        idx_v = scratch[:_NBUF]
        rows_v = scratch[_NBUF:2 * _NBUF]
        g_sem = scratch[2 * _NBUF:3 * _NBUF]
        o_sem = scratch[3 * _NBUF:]

        wid = lax.axis_index("s") * 2 + lax.axis_index("c")
        w_base = wid * _ROWS_PER_W
        iota = lax.iota(jnp.int32, _LANES)

        @pl.when(lax.axis_index("s") == 0)
        def _():
            pltpu.sync_copy(w_hbm, w_sh)

        plsc.subcore_barrier()
        pltpu.sync_copy(feat_hbm.at[pl.ds(w_base, _ROWS_PER_W)], feat_v)

        def start_fetch(k, b):
            for i in range(_CHUNK // _LANES):
                off = k * _CHUNK + i * _LANES
                x = feat_v[pl.ds(off, _LANES)]
                xc = jnp.minimum(jnp.maximum(x, 0.0), float(NUM_BINS - 1))
                r = (xc + _MAGIC) - _MAGIC
                fid = lax.rem(w_base + off + iota, jnp.int32(NUM_FEATURES))
                idx_v[b][pl.ds(i * _LANES, _LANES)] = (
                    fid * NUM_BINS + r.astype(jnp.int32))
            def fire(n, carry):
                v = idx_v[b][pl.ds(n * _LANES, _LANES)]
                for u in range(_LANES):
                    pltpu.async_copy(
                        w_sh.at[pl.ds(v[u], 1)],
                        rows_v[b].at[pl.ds(n * _LANES + u, 1)],
                        g_sem[b])
                return carry

            lax.fori_loop(0, _CHUNK // _LANES, fire, 0)

        def wait_fetch(b):
            pltpu.make_async_copy(
                w_hbm.at[pl.ds(0, _CHUNK)], rows_v[b], g_sem[b]).wait()

        for t in range(_LOOKAHEAD):
            start_fetch(jnp.int32(t), t)

        def round_body(rnd, carry):
            for b in range(_NBUF):
                j = rnd * _NBUF + b
                wait_fetch(b)
                pltpu.async_copy(
                    rows_v[b],
                    out_hbm.at[pl.ds(w_base + j * _CHUNK, _CHUNK)],
                    o_sem[b])
                k = j + _LOOKAHEAD
                b2 = (b + _LOOKAHEAD) % _NBUF

                @pl.when(k < _NCHUNK)
                def _():
                    @pl.when(k >= _NBUF)
                    def _():
                        pltpu.make_async_copy(
                            rows_v[b2],
                            out_hbm.at[
                                pl.ds(w_base + (k - _NBUF) * _CHUNK, _CHUNK)],
                            o_sem[b2]).wait()

                    start_fetch(k, b2)

            return carry

        lax.fori_loop(0, _NCHUNK // _NBUF, round_body, 0)

        for b in range(_NBUF):
            j = _NCHUNK - _NBUF + b
            pltpu.make_async_copy(
                rows_v[b],
                out_hbm.at[pl.ds(w_base + j * _CHUNK, _CHUNK)],
                o_sem[b]).wait()

    return body(feat_flat, w_flat)


def kernel(features, W):
    feat_flat = features.reshape(_ROWS)
    w_flat = W.reshape(NUM_FEATURES * NUM_BINS, EMBED_DIM)
    out = _sc_gather(feat_flat, w_flat)
    return out.reshape(BATCH, NUM_FEATURES * EMBED_DIM)
